# baseline (speedup 1.0000x reference)
import functools

import jax
import jax.numpy as jnp
from jax import lax
from jax.experimental import pallas as pl
from jax.experimental.pallas import tpu as pltpu
from jax.experimental.pallas import tpu_sc as plsc

VOCAB = 30522
HIDDEN = 768
B = 4
S = 2048
TOK = B * S

NC = 2
NS = 16
NW = NC * NS
NSPLIT = 4
SLICE = TOK // NSPLIT
RPW = SLICE // NW
TB = 256


HRPW = RPW // 2


def _gather_rows(ids_h, tok_h, out_h, idx0_v, idx1_v, buf0_v, buf1_v,
                 gsem, osem):
    wid = lax.axis_index("s") * NC + lax.axis_index("c")
    base = wid * RPW
    pltpu.sync_copy(ids_h.at[pl.ds(base, HRPW)], idx0_v)
    g0 = pltpu.async_copy(tok_h.at[idx0_v], buf0_v, gsem)
    pltpu.sync_copy(ids_h.at[pl.ds(base + HRPW, HRPW)], idx1_v)
    g1 = pltpu.async_copy(tok_h.at[idx1_v], buf1_v, gsem)
    g0.wait()
    o0 = pltpu.async_copy(buf0_v, out_h.at[pl.ds(base, HRPW), :], osem)
    g1.wait()
    o1 = pltpu.async_copy(buf1_v, out_h.at[pl.ds(base + HRPW, HRPW), :], osem)
    o0.wait()
    o1.wait()


def _sc_gather(ids_slice, token_table):
    mesh = plsc.VectorSubcoreMesh(core_axis_name="c", subcore_axis_name="s")
    run = pl.kernel(
        _gather_rows,
        out_type=jax.ShapeDtypeStruct((SLICE, HIDDEN), jnp.float32),
        mesh=mesh,
        compiler_params=pltpu.CompilerParams(needs_layout_passes=False),
        scratch_types=[
            pltpu.VMEM((HRPW,), jnp.int32),
            pltpu.VMEM((HRPW,), jnp.int32),
            pltpu.VMEM((HRPW, HIDDEN), jnp.float32),
            pltpu.VMEM((HRPW, HIDDEN), jnp.float32),
            pltpu.SemaphoreType.DMA,
            pltpu.SemaphoreType.DMA,
        ],
    )
    return run(ids_slice, token_table)


def _ln_body(tok_ref, pos_ref, gam_ref, bet_ref, o_ref):
    x = tok_ref[...] + pos_ref[...]
    mean = jnp.mean(x, axis=1, keepdims=True)
    xc = x - mean
    var = jnp.mean(xc * xc, axis=1, keepdims=True)
    rstd = jax.lax.rsqrt(var + 1e-12)
    o_ref[...] = xc * rstd * gam_ref[...][None, :] + bet_ref[...][None, :]


def _tc_layernorm(gathered, pos_table, ln_gamma, ln_beta):
    return pl.pallas_call(
        _ln_body,
        grid=(SLICE // TB,),
        in_specs=[
            pl.BlockSpec((TB, HIDDEN), lambda i: (i, 0)),
            pl.BlockSpec((TB, HIDDEN), lambda i: (i, 0)),
            pl.BlockSpec((HIDDEN,), lambda i: (0,)),
            pl.BlockSpec((HIDDEN,), lambda i: (0,)),
        ],
        out_specs=pl.BlockSpec((TB, HIDDEN), lambda i: (i, 0)),
        out_shape=jax.ShapeDtypeStruct((SLICE, HIDDEN), jnp.float32),
    )(gathered, pos_table, ln_gamma, ln_beta)


@jax.jit
def kernel(input_ids, token_table, pos_table, ln_gamma, ln_beta):
    ids_flat = input_ids.reshape(TOK).astype(jnp.int32)
    gathered = []
    for k in range(NSPLIT):
        ids_k = lax.dynamic_slice_in_dim(ids_flat, k * SLICE, SLICE)
        gathered.append(_sc_gather(ids_k, token_table))
    outs = [_tc_layernorm(g, pos_table, ln_gamma, ln_beta) for g in gathered]
    return jnp.concatenate(outs, axis=0).reshape(B, S, HIDDEN)


# speedup vs baseline: 1.2233x; 1.2233x over previous
import functools

import jax
import jax.numpy as jnp
from jax import lax
from jax.experimental import pallas as pl
from jax.experimental.pallas import tpu as pltpu
from jax.experimental.pallas import tpu_sc as plsc

VOCAB = 30522
HIDDEN = 768
B = 4
S = 2048
TOK = B * S

NC = 2
NS = 16
NW = NC * NS
TPW = TOK // NW
CHW = 32
NCHK = TPW // CHW
TB = 256
SB = S // TB


def _gather_rows(ids_h, tok_h, out_h, idx_v, buf0_v, buf1_v,
                 gsem0, gsem1, osem0, osem1):
    wid = lax.axis_index("s") * NC + lax.axis_index("c")
    base = wid * TPW
    pltpu.sync_copy(ids_h.at[pl.ds(base, TPW)], idx_v)
    bufs = (buf0_v, buf1_v)
    gsems = (gsem0, gsem1)
    osems = (osem0, osem1)
    gd = [None] * NCHK
    od = [None] * NCHK
    gd[0] = pltpu.async_copy(tok_h.at[idx_v.at[pl.ds(0, CHW)]],
                             bufs[0], gsems[0])
    for c in range(NCHK):
        p = c % 2
        if c + 1 < NCHK:
            q = (c + 1) % 2
            if c >= 1:
                od[c - 1].wait()
            gd[c + 1] = pltpu.async_copy(
                tok_h.at[idx_v.at[pl.ds((c + 1) * CHW, CHW)]],
                bufs[q], gsems[q])
        gd[c].wait()
        od[c] = pltpu.async_copy(
            bufs[p], out_h.at[pl.ds(base + c * CHW, CHW), :], osems[p])
    od[NCHK - 2].wait()
    od[NCHK - 1].wait()


def _sc_gather(ids_flat, token_table):
    mesh = plsc.VectorSubcoreMesh(core_axis_name="c", subcore_axis_name="s")
    run = pl.kernel(
        _gather_rows,
        out_type=jax.ShapeDtypeStruct((TOK, HIDDEN), jnp.float32),
        mesh=mesh,
        compiler_params=pltpu.CompilerParams(needs_layout_passes=False),
        scratch_types=[
            pltpu.VMEM((TPW,), jnp.int32),
            pltpu.VMEM((CHW, HIDDEN), jnp.float32),
            pltpu.VMEM((CHW, HIDDEN), jnp.float32),
            pltpu.SemaphoreType.DMA,
            pltpu.SemaphoreType.DMA,
            pltpu.SemaphoreType.DMA,
            pltpu.SemaphoreType.DMA,
        ],
    )
    return run(ids_flat, token_table)


def _ln_body(tok_ref, pos_ref, gam_ref, bet_ref, o_ref):
    x = tok_ref[...] + pos_ref[...]
    mean = jnp.mean(x, axis=1, keepdims=True)
    xc = x - mean
    var = jnp.mean(xc * xc, axis=1, keepdims=True)
    rstd = jax.lax.rsqrt(var + 1e-12)
    o_ref[...] = xc * rstd * gam_ref[...][None, :] + bet_ref[...][None, :]


def _tc_layernorm(gathered, pos_table, ln_gamma, ln_beta):
    return pl.pallas_call(
        _ln_body,
        grid=(TOK // TB,),
        in_specs=[
            pl.BlockSpec((TB, HIDDEN), lambda i: (i, 0)),
            pl.BlockSpec((TB, HIDDEN), lambda i: (i % SB, 0)),
            pl.BlockSpec((HIDDEN,), lambda i: (0,)),
            pl.BlockSpec((HIDDEN,), lambda i: (0,)),
        ],
        out_specs=pl.BlockSpec((TB, HIDDEN), lambda i: (i, 0)),
        out_shape=jax.ShapeDtypeStruct((TOK, HIDDEN), jnp.float32),
    )(gathered, pos_table, ln_gamma, ln_beta)


@jax.jit
def kernel(input_ids, token_table, pos_table, ln_gamma, ln_beta):
    ids_flat = input_ids.reshape(TOK).astype(jnp.int32)
    gathered = _sc_gather(ids_flat, token_table)
    out = _tc_layernorm(gathered, pos_table, ln_gamma, ln_beta)
    return out.reshape(B, S, HIDDEN)


# speedup vs baseline: 1.2367x; 1.0109x over previous
import functools

import jax
import jax.numpy as jnp
from jax import lax
from jax.experimental import pallas as pl
from jax.experimental.pallas import tpu as pltpu
from jax.experimental.pallas import tpu_sc as plsc

VOCAB = 30522
HIDDEN = 768
B = 4
S = 2048
TOK = B * S

NC = 2
NS = 16
NW = NC * NS
TPW = TOK // NW
CHW = 32
NCHK = TPW // CHW
TB = 256
SB = S // TB


def _gather_rows(ids_h, tok_h, out_h, idx_v, buf0_v, buf1_v,
                 gsem0, gsem1, osem0, osem1):
    wid = lax.axis_index("s") * NC + lax.axis_index("c")
    base = wid * TPW
    pltpu.sync_copy(ids_h.at[pl.ds(base, TPW)], idx_v)
    bufs = (buf0_v, buf1_v)
    gsems = (gsem0, gsem1)
    osems = (osem0, osem1)
    gd = [None] * NCHK
    od = [None] * NCHK
    gd[0] = pltpu.async_copy(tok_h.at[idx_v.at[pl.ds(0, CHW)]],
                             bufs[0], gsems[0])
    for c in range(NCHK):
        p = c % 2
        if c + 1 < NCHK:
            q = (c + 1) % 2
            if c >= 1:
                od[c - 1].wait()
            gd[c + 1] = pltpu.async_copy(
                tok_h.at[idx_v.at[pl.ds((c + 1) * CHW, CHW)]],
                bufs[q], gsems[q])
        gd[c].wait()
        od[c] = pltpu.async_copy(
            bufs[p], out_h.at[pl.ds(base + c * CHW, CHW), :], osems[p])
    od[NCHK - 2].wait()
    od[NCHK - 1].wait()


def _sc_gather(ids_flat, token_table):
    mesh = plsc.VectorSubcoreMesh(core_axis_name="c", subcore_axis_name="s")
    run = pl.kernel(
        _gather_rows,
        out_type=jax.ShapeDtypeStruct((TOK, HIDDEN), jnp.float32),
        mesh=mesh,
        compiler_params=pltpu.CompilerParams(needs_layout_passes=False),
        scratch_types=[
            pltpu.VMEM((TPW,), jnp.int32),
            pltpu.VMEM((CHW, HIDDEN), jnp.float32),
            pltpu.VMEM((CHW, HIDDEN), jnp.float32),
            pltpu.SemaphoreType.DMA,
            pltpu.SemaphoreType.DMA,
            pltpu.SemaphoreType.DMA,
            pltpu.SemaphoreType.DMA,
        ],
    )
    return run(ids_flat, token_table)


def _ln_body(tok_ref, pos_ref, gam_ref, bet_ref, o_ref):
    x = tok_ref[...] + pos_ref[...]
    mean = jnp.mean(x, axis=1, keepdims=True)
    xc = x - mean
    var = jnp.mean(xc * xc, axis=1, keepdims=True)
    rstd = jax.lax.rsqrt(var + 1e-12)
    o_ref[...] = xc * rstd * gam_ref[...][None, :] + bet_ref[...][None, :]


def _tc_layernorm(gathered, pos_table, ln_gamma, ln_beta):
    return pl.pallas_call(
        _ln_body,
        grid=(SB, B),
        in_specs=[
            pl.BlockSpec((TB, HIDDEN), lambda j, b: (b * SB + j, 0)),
            pl.BlockSpec((TB, HIDDEN), lambda j, b: (j, 0)),
            pl.BlockSpec((HIDDEN,), lambda j, b: (0,)),
            pl.BlockSpec((HIDDEN,), lambda j, b: (0,)),
        ],
        out_specs=pl.BlockSpec((TB, HIDDEN), lambda j, b: (b * SB + j, 0)),
        out_shape=jax.ShapeDtypeStruct((TOK, HIDDEN), jnp.float32),
    )(gathered, pos_table, ln_gamma, ln_beta)


@jax.jit
def kernel(input_ids, token_table, pos_table, ln_gamma, ln_beta):
    ids_flat = input_ids.reshape(TOK).astype(jnp.int32)
    gathered = _sc_gather(ids_flat, token_table)
    out = _tc_layernorm(gathered, pos_table, ln_gamma, ln_beta)
    return out.reshape(B, S, HIDDEN)


# speedup vs baseline: 1.4235x; 1.1511x over previous
import functools

import jax
import jax.numpy as jnp
from jax import lax
from jax.experimental import pallas as pl
from jax.experimental.pallas import tpu as pltpu
from jax.experimental.pallas import tpu_sc as plsc

VOCAB = 30522
HIDDEN = 768
B = 4
S = 2048
TOK = B * S

NC = 2
NS = 16
NW = NC * NS
TPW = TOK // NW
CHW = 32
NCHK = TPW // CHW
TB = 512
SB = S // TB


def _gather_rows(ids_h, tok_h, out_h, idx_v, buf0_v, buf1_v,
                 gsem0, gsem1, osem0, osem1):
    wid = lax.axis_index("s") * NC + lax.axis_index("c")
    base = wid * TPW
    pltpu.sync_copy(ids_h.at[pl.ds(base, TPW)], idx_v)
    bufs = (buf0_v, buf1_v)
    gsems = (gsem0, gsem1)
    osems = (osem0, osem1)
    gd = [None] * NCHK
    od = [None] * NCHK
    gd[0] = pltpu.async_copy(tok_h.at[idx_v.at[pl.ds(0, CHW)]],
                             bufs[0], gsems[0])
    for c in range(NCHK):
        p = c % 2
        if c + 1 < NCHK:
            q = (c + 1) % 2
            if c >= 1:
                od[c - 1].wait()
            gd[c + 1] = pltpu.async_copy(
                tok_h.at[idx_v.at[pl.ds((c + 1) * CHW, CHW)]],
                bufs[q], gsems[q])
        gd[c].wait()
        od[c] = pltpu.async_copy(
            bufs[p], out_h.at[pl.ds(base + c * CHW, CHW), :], osems[p])
    od[NCHK - 2].wait()
    od[NCHK - 1].wait()


def _sc_gather(ids_flat, token_table):
    mesh = plsc.VectorSubcoreMesh(core_axis_name="c", subcore_axis_name="s")
    run = pl.kernel(
        _gather_rows,
        out_type=jax.ShapeDtypeStruct((TOK, HIDDEN), jnp.float32),
        mesh=mesh,
        compiler_params=pltpu.CompilerParams(needs_layout_passes=False),
        scratch_types=[
            pltpu.VMEM((TPW,), jnp.int32),
            pltpu.VMEM((CHW, HIDDEN), jnp.float32),
            pltpu.VMEM((CHW, HIDDEN), jnp.float32),
            pltpu.SemaphoreType.DMA,
            pltpu.SemaphoreType.DMA,
            pltpu.SemaphoreType.DMA,
            pltpu.SemaphoreType.DMA,
        ],
    )
    return run(ids_flat, token_table)


def _ln_body(tok_ref, pos_ref, gam_ref, bet_ref, o_ref):
    x = tok_ref[...] + pos_ref[...]
    mean = jnp.mean(x, axis=1, keepdims=True)
    xc = x - mean
    var = jnp.mean(xc * xc, axis=1, keepdims=True)
    rstd = jax.lax.rsqrt(var + 1e-12)
    o_ref[...] = xc * rstd * gam_ref[...][None, :] + bet_ref[...][None, :]


def _tc_layernorm(gathered, pos_table, ln_gamma, ln_beta):
    return pl.pallas_call(
        _ln_body,
        grid=(SB, B),
        in_specs=[
            pl.BlockSpec((TB, HIDDEN), lambda j, b: (b * SB + j, 0)),
            pl.BlockSpec((TB, HIDDEN), lambda j, b: (j, 0)),
            pl.BlockSpec((HIDDEN,), lambda j, b: (0,)),
            pl.BlockSpec((HIDDEN,), lambda j, b: (0,)),
        ],
        out_specs=pl.BlockSpec((TB, HIDDEN), lambda j, b: (b * SB + j, 0)),
        out_shape=jax.ShapeDtypeStruct((TOK, HIDDEN), jnp.float32),
    )(gathered, pos_table, ln_gamma, ln_beta)


@jax.jit
def kernel(input_ids, token_table, pos_table, ln_gamma, ln_beta):
    ids_flat = input_ids.reshape(TOK).astype(jnp.int32)
    gathered = _sc_gather(ids_flat, token_table)
    out = _tc_layernorm(gathered, pos_table, ln_gamma, ln_beta)
    return out.reshape(B, S, HIDDEN)
